# baseline (speedup 1.0000x reference)
import functools

import jax
import jax.numpy as jnp
from jax import lax
from jax.experimental import pallas as pl
from jax.experimental.pallas import tpu as pltpu
from jax.experimental.pallas import tpu_sc as plsc

_B = 16384
_NCAT = 20
_NNUM = 6
_V = 100000
_C = 32
_N = _NCAT + _NNUM

_NW = 32
_RPW = _B // _NW
_BC = 64
_NITER = _RPW // _BC


def _embed_sc(idx_w, vals_w, tables_flat, w, bias):
  mesh = plsc.VectorSubcoreMesh(core_axis_name="c", subcore_axis_name="s")

  @functools.partial(
      pl.kernel,
      mesh=mesh,
      out_type=jax.ShapeDtypeStruct((_B * _C * _N,), jnp.float32),
      compiler_params=pltpu.CompilerParams(
          needs_layout_passes=False, use_tc_tiling_on_sc=False),
      scratch_types=[
          pltpu.VMEM((_NCAT, _RPW), jnp.int32),
          pltpu.VMEM((_NNUM * _RPW,), jnp.float32),
          pltpu.VMEM((_NCAT, _BC, _C), jnp.float32),
          pltpu.VMEM((_BC * _C * _N,), jnp.float32),
          pltpu.VMEM((_NNUM, _C), jnp.float32),
          pltpu.VMEM((_NNUM, _C), jnp.float32),
          pltpu.SemaphoreType.DMA,
      ],
  )
  def k(idx_hbm, vals_hbm, tab_hbm, w_hbm, b_hbm, out_hbm,
        idxb, valb, gbuf, obuf, wbuf, bbuf, sem):
    wid = lax.axis_index("s") * 2 + lax.axis_index("c")
    iota = lax.iota(jnp.int32, 16)
    iota26 = iota * _N

    pltpu.sync_copy(w_hbm, wbuf)
    pltpu.sync_copy(b_hbm, bbuf)
    pltpu.sync_copy(idx_hbm.at[wid], idxb)
    pltpu.sync_copy(vals_hbm.at[wid], valb)

    def chunk_body(it, _):
      off = it * _BC
      cps = [
          pltpu.async_copy(
              tab_hbm.at[idxb.at[f, pl.ds(off, _BC)]], gbuf.at[f], sem)
          for f in range(_NCAT)
      ]
      for cp in cps:
        cp.wait()

      def row_body(i, _):
        rbase = i * (_C * _N)
        for f in range(_NCAT):
          for h in range(2):
            v = gbuf[f, i, pl.ds(h * 16, 16)]
            pos = iota26 + (rbase + f + h * 16 * _N)
            plsc.store_scatter(obuf, [pos], v)
        gi = jnp.full((16,), off + i, jnp.int32)
        for j in range(_NNUM):
          s = plsc.load_gather(valb, [gi + j * _RPW])
          for h in range(2):
            v = s * wbuf[j, pl.ds(h * 16, 16)] + bbuf[j, pl.ds(h * 16, 16)]
            pos = iota26 + (rbase + _NCAT + j + h * 16 * _N)
            plsc.store_scatter(obuf, [pos], v)
        return _

      lax.fori_loop(0, _BC, row_body, None)
      pltpu.sync_copy(
          obuf, out_hbm.at[pl.ds((wid * _RPW + off) * _C * _N, _BC * _C * _N)])
      return _

    lax.fori_loop(0, _NITER, chunk_body, None)

  return k(idx_w, vals_w, tables_flat, w, bias)


def kernel(x_tensor, tables, W, b):
  idx_t = (x_tensor[:, :_NCAT].astype(jnp.int32)
           + jnp.arange(_NCAT, dtype=jnp.int32)[None, :] * _V).T
  idx_w = idx_t.reshape(_NCAT, _NW, _RPW).transpose(1, 0, 2)
  vals_w = (x_tensor[:, _NCAT:].T.reshape(_NNUM, _NW, _RPW)
            .transpose(1, 0, 2).reshape(_NW, _NNUM * _RPW))
  tables_flat = tables.reshape(_NCAT * _V, _C)
  out_flat = _embed_sc(idx_w, vals_w, tables_flat, W, b)
  return out_flat.reshape(_B, _C, _N)


# speedup vs baseline: 1.0065x; 1.0065x over previous
import functools

import jax
import jax.numpy as jnp
from jax import lax
from jax.experimental import pallas as pl
from jax.experimental.pallas import tpu as pltpu
from jax.experimental.pallas import tpu_sc as plsc

_B = 16384
_NCAT = 20
_NNUM = 6
_V = 100000
_C = 32
_N = _NCAT + _NNUM

_NW = 32
_RPW = _B // _NW
_BC = 64
_NITER = _RPW // _BC


def _embed_sc(idx_w, vals_w, tables_flat, w, bias):
  mesh = plsc.VectorSubcoreMesh(core_axis_name="c", subcore_axis_name="s")

  @functools.partial(
      pl.kernel,
      mesh=mesh,
      out_type=jax.ShapeDtypeStruct((_B * _C * _N,), jnp.float32),
      compiler_params=pltpu.CompilerParams(
          needs_layout_passes=False, use_tc_tiling_on_sc=False),
      scratch_types=[
          pltpu.VMEM((_NCAT, _RPW), jnp.int32),
          pltpu.VMEM((_NNUM * _RPW,), jnp.float32),
          pltpu.VMEM((2, _BC, _C), jnp.float32),
          pltpu.VMEM((_BC * _C * _N,), jnp.float32),
          pltpu.VMEM((_NNUM, _C), jnp.float32),
          pltpu.VMEM((_NNUM, _C), jnp.float32),
          pltpu.SemaphoreType.DMA,
          pltpu.SemaphoreType.DMA,
      ],
  )
  def k(idx_hbm, vals_hbm, tab_hbm, w_hbm, b_hbm, out_hbm,
        idxb, valb, gbuf, obuf, wbuf, bbuf, sem0, sem1):
    wid = lax.axis_index("s") * 2 + lax.axis_index("c")
    iota = lax.iota(jnp.int32, 16)
    col0 = iota * _N
    col1 = (iota + 16) * _N

    pltpu.sync_copy(w_hbm, wbuf)
    pltpu.sync_copy(b_hbm, bbuf)
    pltpu.sync_copy(idx_hbm.at[wid], idxb)
    pltpu.sync_copy(vals_hbm.at[wid], valb)
    sems = (sem0, sem1)

    def chunk_body(it, _):
      off = it * _BC

      def fire(f):
        return pltpu.async_copy(
            tab_hbm.at[idxb.at[f, pl.ds(off, _BC)]], gbuf.at[f % 2],
            sems[f % 2])

      cp = fire(0)
      for f in range(_NCAT):
        nxt = fire(f + 1) if f + 1 < _NCAT else None
        cp.wait()

        def row_body(i, _, f=f):
          rb = i * (_C * _N) + f
          plsc.store_scatter(obuf, [col0 + rb], gbuf[f % 2, i, pl.ds(0, 16)])
          plsc.store_scatter(obuf, [col1 + rb], gbuf[f % 2, i, pl.ds(16, 16)])
          return _

        lax.fori_loop(0, _BC, row_body, None)
        cp = nxt

      def num_body(i, _):
        gi = jnp.full((16,), off + i, jnp.int32)
        for j in range(_NNUM):
          s = plsc.load_gather(valb, [gi + j * _RPW])
          rb = i * (_C * _N) + _NCAT + j
          v0 = s * wbuf[j, pl.ds(0, 16)] + bbuf[j, pl.ds(0, 16)]
          v1 = s * wbuf[j, pl.ds(16, 16)] + bbuf[j, pl.ds(16, 16)]
          plsc.store_scatter(obuf, [col0 + rb], v0)
          plsc.store_scatter(obuf, [col1 + rb], v1)
        return _

      lax.fori_loop(0, _BC, num_body, None)
      pltpu.sync_copy(
          obuf, out_hbm.at[pl.ds((wid * _RPW + off) * _C * _N, _BC * _C * _N)])
      return _

    lax.fori_loop(0, _NITER, chunk_body, None)

  return k(idx_w, vals_w, tables_flat, w, bias)


def kernel(x_tensor, tables, W, b):
  idx_t = (x_tensor[:, :_NCAT].astype(jnp.int32)
           + jnp.arange(_NCAT, dtype=jnp.int32)[None, :] * _V).T
  idx_w = idx_t.reshape(_NCAT, _NW, _RPW).transpose(1, 0, 2)
  vals_w = (x_tensor[:, _NCAT:].T.reshape(_NNUM, _NW, _RPW)
            .transpose(1, 0, 2).reshape(_NW, _NNUM * _RPW))
  tables_flat = tables.reshape(_NCAT * _V, _C)
  out_flat = _embed_sc(idx_w, vals_w, tables_flat, W, b)
  return out_flat.reshape(_B, _C, _N)


# speedup vs baseline: 1.0070x; 1.0005x over previous
import functools

import jax
import jax.numpy as jnp
from jax import lax
from jax.experimental import pallas as pl
from jax.experimental.pallas import tpu as pltpu
from jax.experimental.pallas import tpu_sc as plsc

_B = 16384
_NCAT = 20
_NNUM = 6
_V = 100000
_C = 32
_N = _NCAT + _NNUM

_NW = 32
_RPW = _B // _NW
_BC = 64
_NITER = _RPW // _BC


def _embed_sc(idx_w, vals_w, tables_flat, w, bias):
  mesh = plsc.VectorSubcoreMesh(core_axis_name="c", subcore_axis_name="s")

  @functools.partial(
      pl.kernel,
      mesh=mesh,
      out_type=jax.ShapeDtypeStruct((_B * _C * _N,), jnp.float32),
      compiler_params=pltpu.CompilerParams(
          needs_layout_passes=False, use_tc_tiling_on_sc=False),
      scratch_types=[
          pltpu.VMEM((_NCAT, _RPW), jnp.int32),
          pltpu.VMEM((_NNUM * _RPW,), jnp.float32),
          pltpu.VMEM((2, _BC, _C), jnp.float32),
          pltpu.VMEM((_BC * _C * _N,), jnp.float32),
          pltpu.VMEM((_NNUM, _C), jnp.float32),
          pltpu.VMEM((_NNUM, _C), jnp.float32),
          pltpu.SemaphoreType.DMA,
          pltpu.SemaphoreType.DMA,
      ],
  )
  def k(idx_hbm, vals_hbm, tab_hbm, w_hbm, b_hbm, out_hbm,
        idxb, valb, gbuf, obuf, wbuf, bbuf, sem0, sem1):
    wid = lax.axis_index("s") * 2 + lax.axis_index("c")
    iota = lax.iota(jnp.int32, 16)
    col0 = iota * _N
    col1 = (iota + 16) * _N

    pltpu.sync_copy(w_hbm, wbuf)
    pltpu.sync_copy(b_hbm, bbuf)
    pltpu.sync_copy(idx_hbm.at[wid], idxb)
    pltpu.sync_copy(vals_hbm.at[wid], valb)
    sems = (sem0, sem1)

    def chunk_body(it, _):
      off = it * _BC

      def fire(f):
        return pltpu.async_copy(
            tab_hbm.at[f].at[idxb.at[f, pl.ds(off, _BC)]], gbuf.at[f % 2],
            sems[f % 2])

      cp = fire(0)
      for f in range(_NCAT):
        nxt = fire(f + 1) if f + 1 < _NCAT else None
        cp.wait()

        def row_body(i, _, f=f):
          rb = i * (_C * _N) + f
          plsc.store_scatter(obuf, [col0 + rb], gbuf[f % 2, i, pl.ds(0, 16)])
          plsc.store_scatter(obuf, [col1 + rb], gbuf[f % 2, i, pl.ds(16, 16)])
          return _

        lax.fori_loop(0, _BC, row_body, None)
        cp = nxt

      def num_body(i, _):
        gi = jnp.full((16,), off + i, jnp.int32)
        for j in range(_NNUM):
          s = plsc.load_gather(valb, [gi + j * _RPW])
          rb = i * (_C * _N) + _NCAT + j
          v0 = s * wbuf[j, pl.ds(0, 16)] + bbuf[j, pl.ds(0, 16)]
          v1 = s * wbuf[j, pl.ds(16, 16)] + bbuf[j, pl.ds(16, 16)]
          plsc.store_scatter(obuf, [col0 + rb], v0)
          plsc.store_scatter(obuf, [col1 + rb], v1)
        return _

      lax.fori_loop(0, _BC, num_body, None)
      pltpu.sync_copy(
          obuf, out_hbm.at[pl.ds((wid * _RPW + off) * _C * _N, _BC * _C * _N)])
      return _

    lax.fori_loop(0, _NITER, chunk_body, None)

  return k(idx_w, vals_w, tables_flat, w, bias)


def kernel(x_tensor, tables, W, b):
  idx_t = x_tensor[:, :_NCAT].astype(jnp.int32).T
  idx_w = idx_t.reshape(_NCAT, _NW, _RPW).transpose(1, 0, 2)
  vals_w = (x_tensor[:, _NCAT:].T.reshape(_NNUM, _NW, _RPW)
            .transpose(1, 0, 2).reshape(_NW, _NNUM * _RPW))
  out_flat = _embed_sc(idx_w, vals_w, tables, W, b)
  return out_flat.reshape(_B, _C, _N)
